# baseline (speedup 1.0000x reference)
import functools

import jax
import jax.numpy as jnp
from jax import lax
from jax.experimental import pallas as pl
from jax.experimental.pallas import tpu as pltpu
from jax.experimental.pallas import tpu_sc as plsc

N = 10000
E = 320000
D = 128
DE = 16

NC = 2
NS = 16
L = 16
NW = NC * NS
DPT = D // NW
C = 1280
NCHUNK = E // C


def _edge_proj_body(weT_ref, eaT_ref, be_ref, out_ref):
    out_ref[...] = lax.dot_general(
        weT_ref[...], eaT_ref[...], (((1,), (0,)), ((), ())),
        preferred_element_type=jnp.float32) + be_ref[...]


def _edge_proj(eaT, WeT, be2d):
    ce = 2560
    grid = E // ce
    return pl.pallas_call(
        _edge_proj_body,
        grid=(grid,),
        in_specs=[
            pl.BlockSpec((D, DE), lambda i: (0, 0)),
            pl.BlockSpec((DE, ce), lambda i: (0, i)),
            pl.BlockSpec((D, 1), lambda i: (0, 0)),
        ],
        out_specs=pl.BlockSpec((D, ce), lambda i: (0, i)),
        out_shape=jax.ShapeDtypeStruct((D, E), jnp.float32),
    )(WeT, eaT, be2d)


def _xpose_body(x_ref, out_ref):
    out_ref[...] = x_ref[...].T


def _xpose(x):
    return pl.pallas_call(
        _xpose_body,
        grid=(1,),
        in_specs=[pl.BlockSpec((N, D), lambda i: (0, 0))],
        out_specs=pl.BlockSpec((D, N), lambda i: (0, 0)),
        out_shape=jax.ShapeDtypeStruct((D, N), jnp.float32),
    )(x)


def _sc_body(xT_hbm, ei_hbm, eT_hbm, out_hbm, xv, av, ev, sv, es0, es1,
             is0, is1, xsem):
    half = lax.axis_index("c")
    pid = lax.axis_index("s")
    wid = pid * NC + half
    esems = (es0, es1)
    isems = (is0, is1)

    xcp = pltpu.make_async_copy(xT_hbm.at[wid], xv, xsem)
    xcp.start()

    def e_copy(ci, b):
        return pltpu.make_async_copy(
            eT_hbm.at[pid, :, pl.ds(ci * C, C)], ev.at[b], esems[b])

    def i_copy(ci, b):
        return pltpu.make_async_copy(
            ei_hbm.at[:, pl.ds(ci * C, C)], sv.at[b], isems[b])

    for b in range(2):
        e_copy(b, b).start()
        i_copy(b, b).start()

    zeros = jnp.zeros((L,), jnp.float32)

    @plsc.parallel_loop(0, N // L, unroll=8)
    def _zb(i):
        for d in range(DPT):
            av[d, pl.ds(i * L, L)] = zeros

    xcp.wait()

    rbase = DPT * half
    dfull = [jnp.full((L,), d, jnp.int32) for d in range(DPT)]

    def outer(j, carry):
        for b in range(2):
            ci = 2 * j + b
            e_copy(ci, b).wait()
            i_copy(ci, b).wait()

            @plsc.parallel_loop(0, C // L, unroll=8)
            def _gb(g):
                off = g * L
                s = sv[b, 0, pl.ds(off, L)]
                t = sv[b, 1, pl.ds(off, L)]
                for d in range(DPT):
                    evd = ev[b, rbase + d, pl.ds(off, L)]
                    xg = plsc.load_gather(xv, [dfull[d], s])
                    m = jnp.maximum(xg + evd, 0.0)
                    plsc.addupdate_scatter(av, [dfull[d], t], m)

            nci = ci + 2

            @pl.when(nci < NCHUNK)
            def _():
                e_copy(nci, b).start()
                i_copy(nci, b).start()
        return carry

    lax.fori_loop(0, NCHUNK // 2, outer, 0)
    pltpu.sync_copy(av, out_hbm.at[wid])


def _sc_aggr(xT, ei, eT):
    mesh = plsc.VectorSubcoreMesh(core_axis_name="c", subcore_axis_name="s")
    f = functools.partial(
        pl.kernel,
        out_type=jax.ShapeDtypeStruct((NW, DPT, N), jnp.float32),
        mesh=mesh,
        compiler_params=pltpu.CompilerParams(needs_layout_passes=False),
        scratch_types=[
            pltpu.VMEM((DPT, N), jnp.float32),
            pltpu.VMEM((DPT, N), jnp.float32),
            pltpu.VMEM((2, 2 * DPT, C), jnp.float32),
            pltpu.VMEM((2, 2, C), jnp.int32),
            pltpu.SemaphoreType.DMA,
            pltpu.SemaphoreType.DMA,
            pltpu.SemaphoreType.DMA,
            pltpu.SemaphoreType.DMA,
            pltpu.SemaphoreType.DMA,
        ],
    )(_sc_body)
    return f(xT.reshape(NW, DPT, N), ei, eT.reshape(NS, NC * DPT, E))


def _mlp_body(x_ref, at_ref, w1_ref, b1_ref, w2_ref, b2_ref, g_ref, bt_ref,
              o_ref):
    x = x_ref[...]
    h = x + at_ref[...].T
    h1 = jnp.maximum(
        jnp.dot(h, w1_ref[...], preferred_element_type=jnp.float32)
        + b1_ref[...], 0.0)
    h2 = jnp.dot(h1, w2_ref[...], preferred_element_type=jnp.float32) \
        + b2_ref[...]
    y = jnp.maximum(h2, 0.0) + x
    mu = jnp.mean(y, axis=-1, keepdims=True)
    yc = y - mu
    var = jnp.mean(yc * yc, axis=-1, keepdims=True)
    o_ref[...] = yc * lax.rsqrt(var + 1e-5) * g_ref[...] + bt_ref[...]


def _mlp(x, aggrT, W1, b1, W2, b2, gamma, beta):
    bn = 1024
    grid = pl.cdiv(N, bn)
    return pl.pallas_call(
        _mlp_body,
        grid=(grid,),
        in_specs=[
            pl.BlockSpec((bn, D), lambda i: (i, 0)),
            pl.BlockSpec((D, bn), lambda i: (0, i)),
            pl.BlockSpec((D, D), lambda i: (0, 0)),
            pl.BlockSpec((1, D), lambda i: (0, 0)),
            pl.BlockSpec((D, D), lambda i: (0, 0)),
            pl.BlockSpec((1, D), lambda i: (0, 0)),
            pl.BlockSpec((1, D), lambda i: (0, 0)),
            pl.BlockSpec((1, D), lambda i: (0, 0)),
        ],
        out_specs=pl.BlockSpec((bn, D), lambda i: (i, 0)),
        out_shape=jax.ShapeDtypeStruct((N, D), jnp.float32),
    )(x, aggrT, W1, b1, W2, b2, gamma, beta)


def kernel(x, edge_index, edge_attr, We, be, W1, b1, W2, b2, gamma, beta):
    ei = edge_index.astype(jnp.int32)
    eT = _edge_proj(edge_attr.T, We.T, be.reshape(D, 1))
    xT = _xpose(x)
    aggrT = _sc_aggr(xT, ei, eT)
    return _mlp(x, aggrT.reshape(D, N), W1, b1.reshape(1, D), W2,
                b2.reshape(1, D), gamma.reshape(1, D), beta.reshape(1, D))


# speedup vs baseline: 1.0433x; 1.0433x over previous
import functools

import jax
import jax.numpy as jnp
from jax import lax
from jax.experimental import pallas as pl
from jax.experimental.pallas import tpu as pltpu
from jax.experimental.pallas import tpu_sc as plsc

N = 10000
E = 320000
D = 128
DE = 16

NC = 2
NS = 16
L = 16
NW = NC * NS
DPT = D // NW
C = 640
NCHUNK = E // C


def _edge_proj_body(weT_ref, eaT_ref, be_ref, out_ref):
    out_ref[...] = lax.dot_general(
        weT_ref[...], eaT_ref[...], (((1,), (0,)), ((), ())),
        preferred_element_type=jnp.float32) + be_ref[...]


def _edge_proj(eaT, WeT, be2d):
    ce = 2560
    grid = E // ce
    return pl.pallas_call(
        _edge_proj_body,
        grid=(grid,),
        in_specs=[
            pl.BlockSpec((D, DE), lambda i: (0, 0)),
            pl.BlockSpec((DE, ce), lambda i: (0, i)),
            pl.BlockSpec((D, 1), lambda i: (0, 0)),
        ],
        out_specs=pl.BlockSpec((D, ce), lambda i: (0, i)),
        out_shape=jax.ShapeDtypeStruct((D, E), jnp.float32),
    )(WeT, eaT, be2d)


def _xpose_body(x_ref, out_ref):
    out_ref[...] = x_ref[...].T


def _xpose(x):
    return pl.pallas_call(
        _xpose_body,
        grid=(1,),
        in_specs=[pl.BlockSpec((N, D), lambda i: (0, 0))],
        out_specs=pl.BlockSpec((D, N), lambda i: (0, 0)),
        out_shape=jax.ShapeDtypeStruct((D, N), jnp.float32),
    )(x)


NBUF = 5


def _sc_body(xT_hbm, ei_hbm, eT_hbm, out_hbm, xv, av, ev, sv, es0, es1, es2,
             es3, es4, is0, is1, is2, is3, is4, xsem):
    half = lax.axis_index("c")
    pid = lax.axis_index("s")
    wid = pid * NC + half
    esems = (es0, es1, es2, es3, es4)
    isems = (is0, is1, is2, is3, is4)

    xcp = pltpu.make_async_copy(xT_hbm.at[wid], xv, xsem)
    xcp.start()

    def e_copy(ci, b):
        return pltpu.make_async_copy(
            eT_hbm.at[pid, :, pl.ds(ci * C, C)], ev.at[b], esems[b])

    def i_copy(ci, b):
        return pltpu.make_async_copy(
            ei_hbm.at[:, pl.ds(ci * C, C)], sv.at[b], isems[b])

    for b in range(NBUF):
        e_copy(b, b).start()
        i_copy(b, b).start()

    zeros = jnp.zeros((L,), jnp.float32)

    @plsc.parallel_loop(0, N // L, unroll=8)
    def _zb(i):
        for d in range(DPT):
            av[d, pl.ds(i * L, L)] = zeros

    xcp.wait()

    rbase = DPT * half
    dfull = [jnp.full((L,), d, jnp.int32) for d in range(DPT)]

    def outer(j, carry):
        for b in range(NBUF):
            ci = NBUF * j + b
            e_copy(ci, b).wait()
            i_copy(ci, b).wait()

            @plsc.parallel_loop(0, C // L, unroll=4)
            def _gb(g):
                off = g * L
                s = sv[b, 0, pl.ds(off, L)]
                t = sv[b, 1, pl.ds(off, L)]
                for d in range(DPT):
                    evd = ev[b, rbase + d, pl.ds(off, L)]
                    xg = plsc.load_gather(xv, [dfull[d], s])
                    m = jnp.maximum(xg + evd, 0.0)
                    plsc.addupdate_scatter(av, [dfull[d], t], m)

            nci = ci + NBUF

            @pl.when(nci < NCHUNK)
            def _():
                e_copy(nci, b).start()
                i_copy(nci, b).start()
        return carry

    lax.fori_loop(0, NCHUNK // NBUF, outer, 0)
    pltpu.sync_copy(av, out_hbm.at[wid])


def _sc_aggr(xT, ei, eT):
    mesh = plsc.VectorSubcoreMesh(core_axis_name="c", subcore_axis_name="s")
    f = functools.partial(
        pl.kernel,
        out_type=jax.ShapeDtypeStruct((NW, DPT, N), jnp.float32),
        mesh=mesh,
        compiler_params=pltpu.CompilerParams(needs_layout_passes=False),
        scratch_types=[
            pltpu.VMEM((DPT, N), jnp.float32),
            pltpu.VMEM((DPT, N), jnp.float32),
            pltpu.VMEM((NBUF, 2 * DPT, C), jnp.float32),
            pltpu.VMEM((NBUF, 2, C), jnp.int32),
            pltpu.SemaphoreType.DMA,
            pltpu.SemaphoreType.DMA,
            pltpu.SemaphoreType.DMA,
            pltpu.SemaphoreType.DMA,
            pltpu.SemaphoreType.DMA,
            pltpu.SemaphoreType.DMA,
            pltpu.SemaphoreType.DMA,
            pltpu.SemaphoreType.DMA,
            pltpu.SemaphoreType.DMA,
            pltpu.SemaphoreType.DMA,
            pltpu.SemaphoreType.DMA,
        ],
    )(_sc_body)
    return f(xT.reshape(NW, DPT, N), ei, eT.reshape(NS, NC * DPT, E))


def _mlp_body(x_ref, at_ref, w1_ref, b1_ref, w2_ref, b2_ref, g_ref, bt_ref,
              o_ref):
    x = x_ref[...]
    h = x + at_ref[...].T
    h1 = jnp.maximum(
        jnp.dot(h, w1_ref[...], preferred_element_type=jnp.float32)
        + b1_ref[...], 0.0)
    h2 = jnp.dot(h1, w2_ref[...], preferred_element_type=jnp.float32) \
        + b2_ref[...]
    y = jnp.maximum(h2, 0.0) + x
    mu = jnp.mean(y, axis=-1, keepdims=True)
    yc = y - mu
    var = jnp.mean(yc * yc, axis=-1, keepdims=True)
    o_ref[...] = yc * lax.rsqrt(var + 1e-5) * g_ref[...] + bt_ref[...]


def _mlp(x, aggrT, W1, b1, W2, b2, gamma, beta):
    bn = 1024
    grid = pl.cdiv(N, bn)
    return pl.pallas_call(
        _mlp_body,
        grid=(grid,),
        in_specs=[
            pl.BlockSpec((bn, D), lambda i: (i, 0)),
            pl.BlockSpec((D, bn), lambda i: (0, i)),
            pl.BlockSpec((D, D), lambda i: (0, 0)),
            pl.BlockSpec((1, D), lambda i: (0, 0)),
            pl.BlockSpec((D, D), lambda i: (0, 0)),
            pl.BlockSpec((1, D), lambda i: (0, 0)),
            pl.BlockSpec((1, D), lambda i: (0, 0)),
            pl.BlockSpec((1, D), lambda i: (0, 0)),
        ],
        out_specs=pl.BlockSpec((bn, D), lambda i: (i, 0)),
        out_shape=jax.ShapeDtypeStruct((N, D), jnp.float32),
    )(x, aggrT, W1, b1, W2, b2, gamma, beta)


def kernel(x, edge_index, edge_attr, We, be, W1, b1, W2, b2, gamma, beta):
    ei = edge_index.astype(jnp.int32)
    eT = _edge_proj(edge_attr.T, We.T, be.reshape(D, 1))
    xT = _xpose(x)
    aggrT = _sc_aggr(xT, ei, eT)
    return _mlp(x, aggrT.reshape(D, N), W1, b1.reshape(1, D), W2,
                b2.reshape(1, D), gamma.reshape(1, D), beta.reshape(1, D))


# speedup vs baseline: 1.0437x; 1.0004x over previous
import functools

import jax
import jax.numpy as jnp
from jax import lax
from jax.experimental import pallas as pl
from jax.experimental.pallas import tpu as pltpu
from jax.experimental.pallas import tpu_sc as plsc

N = 10000
E = 320000
D = 128
DE = 16

NC = 2
NS = 16
L = 16
NW = NC * NS
DPT = D // NW
C = 640
NCHUNK = E // C


def _edge_proj_body(weT_ref, eaT_ref, be_ref, out_ref):
    out_ref[...] = lax.dot_general(
        weT_ref[...].astype(jnp.bfloat16), eaT_ref[...].astype(jnp.bfloat16),
        (((1,), (0,)), ((), ())),
        preferred_element_type=jnp.float32) + be_ref[...]


def _edge_proj(eaT, WeT, be2d):
    ce = 2560
    grid = E // ce
    return pl.pallas_call(
        _edge_proj_body,
        grid=(grid,),
        in_specs=[
            pl.BlockSpec((D, DE), lambda i: (0, 0)),
            pl.BlockSpec((DE, ce), lambda i: (0, i)),
            pl.BlockSpec((D, 1), lambda i: (0, 0)),
        ],
        out_specs=pl.BlockSpec((D, ce), lambda i: (0, i)),
        out_shape=jax.ShapeDtypeStruct((D, E), jnp.float32),
    )(WeT, eaT, be2d)


def _xpose_body(x_ref, out_ref):
    out_ref[...] = x_ref[...].T


def _xpose(x):
    return pl.pallas_call(
        _xpose_body,
        grid=(1,),
        in_specs=[pl.BlockSpec((N, D), lambda i: (0, 0))],
        out_specs=pl.BlockSpec((D, N), lambda i: (0, 0)),
        out_shape=jax.ShapeDtypeStruct((D, N), jnp.float32),
    )(x)


NBUF = 5


def _sc_body(xT_hbm, ei_hbm, eT_hbm, out_hbm, xv, av, ev, sv, es0, es1, es2,
             es3, es4, is0, is1, is2, is3, is4, xsem):
    half = lax.axis_index("c")
    pid = lax.axis_index("s")
    wid = pid * NC + half
    esems = (es0, es1, es2, es3, es4)
    isems = (is0, is1, is2, is3, is4)

    xcp = pltpu.make_async_copy(xT_hbm.at[wid], xv, xsem)
    xcp.start()

    def e_copy(ci, b):
        return pltpu.make_async_copy(
            eT_hbm.at[pid, :, pl.ds(ci * C, C)], ev.at[b], esems[b])

    def i_copy(ci, b):
        return pltpu.make_async_copy(
            ei_hbm.at[:, pl.ds(ci * C, C)], sv.at[b], isems[b])

    for b in range(NBUF):
        e_copy(b, b).start()
        i_copy(b, b).start()

    zeros = jnp.zeros((L,), jnp.float32)

    @plsc.parallel_loop(0, N // L, unroll=8)
    def _zb(i):
        for d in range(DPT):
            av[d, pl.ds(i * L, L)] = zeros

    xcp.wait()

    rbase = DPT * half
    dfull = [jnp.full((L,), d, jnp.int32) for d in range(DPT)]

    def outer(j, carry):
        for b in range(NBUF):
            ci = NBUF * j + b
            e_copy(ci, b).wait()
            i_copy(ci, b).wait()

            @plsc.parallel_loop(0, C // L, unroll=4)
            def _gb(g):
                off = g * L
                s = sv[b, 0, pl.ds(off, L)]
                t = sv[b, 1, pl.ds(off, L)]
                for d in range(DPT):
                    evd = ev[b, rbase + d, pl.ds(off, L)]
                    xg = plsc.load_gather(xv, [dfull[d], s])
                    m = jnp.maximum(xg + evd, 0.0)
                    plsc.addupdate_scatter(av, [dfull[d], t], m)

            nci = ci + NBUF

            @pl.when(nci < NCHUNK)
            def _():
                e_copy(nci, b).start()
                i_copy(nci, b).start()
        return carry

    lax.fori_loop(0, NCHUNK // NBUF, outer, 0)
    pltpu.sync_copy(av, out_hbm.at[wid])


def _sc_aggr(xT, ei, eT):
    mesh = plsc.VectorSubcoreMesh(core_axis_name="c", subcore_axis_name="s")
    f = functools.partial(
        pl.kernel,
        out_type=jax.ShapeDtypeStruct((NW, DPT, N), jnp.float32),
        mesh=mesh,
        compiler_params=pltpu.CompilerParams(needs_layout_passes=False),
        scratch_types=[
            pltpu.VMEM((DPT, N), jnp.float32),
            pltpu.VMEM((DPT, N), jnp.float32),
            pltpu.VMEM((NBUF, 2 * DPT, C), jnp.float32),
            pltpu.VMEM((NBUF, 2, C), jnp.int32),
            pltpu.SemaphoreType.DMA,
            pltpu.SemaphoreType.DMA,
            pltpu.SemaphoreType.DMA,
            pltpu.SemaphoreType.DMA,
            pltpu.SemaphoreType.DMA,
            pltpu.SemaphoreType.DMA,
            pltpu.SemaphoreType.DMA,
            pltpu.SemaphoreType.DMA,
            pltpu.SemaphoreType.DMA,
            pltpu.SemaphoreType.DMA,
            pltpu.SemaphoreType.DMA,
        ],
    )(_sc_body)
    return f(xT.reshape(NW, DPT, N), ei, eT.reshape(NS, NC * DPT, E))


def _mlp_body(x_ref, at_ref, w1_ref, b1_ref, w2_ref, b2_ref, g_ref, bt_ref,
              o_ref):
    x = x_ref[...]
    h = x + at_ref[...].T
    h1 = jnp.maximum(
        jnp.dot(h, w1_ref[...], preferred_element_type=jnp.float32)
        + b1_ref[...], 0.0)
    h2 = jnp.dot(h1, w2_ref[...], preferred_element_type=jnp.float32) \
        + b2_ref[...]
    y = jnp.maximum(h2, 0.0) + x
    mu = jnp.mean(y, axis=-1, keepdims=True)
    yc = y - mu
    var = jnp.mean(yc * yc, axis=-1, keepdims=True)
    o_ref[...] = yc * lax.rsqrt(var + 1e-5) * g_ref[...] + bt_ref[...]


def _mlp(x, aggrT, W1, b1, W2, b2, gamma, beta):
    bn = 1024
    grid = pl.cdiv(N, bn)
    return pl.pallas_call(
        _mlp_body,
        grid=(grid,),
        in_specs=[
            pl.BlockSpec((bn, D), lambda i: (i, 0)),
            pl.BlockSpec((D, bn), lambda i: (0, i)),
            pl.BlockSpec((D, D), lambda i: (0, 0)),
            pl.BlockSpec((1, D), lambda i: (0, 0)),
            pl.BlockSpec((D, D), lambda i: (0, 0)),
            pl.BlockSpec((1, D), lambda i: (0, 0)),
            pl.BlockSpec((1, D), lambda i: (0, 0)),
            pl.BlockSpec((1, D), lambda i: (0, 0)),
        ],
        out_specs=pl.BlockSpec((bn, D), lambda i: (i, 0)),
        out_shape=jax.ShapeDtypeStruct((N, D), jnp.float32),
    )(x, aggrT, W1, b1, W2, b2, gamma, beta)


def kernel(x, edge_index, edge_attr, We, be, W1, b1, W2, b2, gamma, beta):
    ei = edge_index.astype(jnp.int32)
    eT = _edge_proj(edge_attr.T, We.T, be.reshape(D, 1))
    xT = _xpose(x)
    aggrT = _sc_aggr(xT, ei, eT)
    return _mlp(x, aggrT.reshape(D, N), W1, b1.reshape(1, D), W2,
                b2.reshape(1, D), gamma.reshape(1, D), beta.reshape(1, D))


# speedup vs baseline: 1.1235x; 1.0764x over previous
import functools

import jax
import jax.numpy as jnp
from jax import lax
from jax.experimental import pallas as pl
from jax.experimental.pallas import tpu as pltpu
from jax.experimental.pallas import tpu_sc as plsc

N = 10000
E = 320000
D = 128
DE = 16

NC = 2
NS = 16
L = 16
NW = NC * NS
DPT = D // NW
C = 640
NCHUNK = E // C


EH = E // 2


def _edge_proj_body(weT_ref, eaT_ref, be_ref, out_ref):
    out_ref[...] = lax.dot_general(
        weT_ref[...], eaT_ref[...], (((1,), (0,)), ((), ())),
        preferred_element_type=jnp.float32) + be_ref[...]


def _edge_proj(eaT, WeT, be2d, phase):
    ce = 3200
    grid = EH // ce
    off = phase * (EH // ce)
    return pl.pallas_call(
        _edge_proj_body,
        grid=(grid,),
        in_specs=[
            pl.BlockSpec((D, DE), lambda i: (0, 0)),
            pl.BlockSpec((DE, ce), lambda i: (0, i + off)),
            pl.BlockSpec((D, 1), lambda i: (0, 0)),
        ],
        out_specs=pl.BlockSpec((D, ce), lambda i: (0, i)),
        out_shape=jax.ShapeDtypeStruct((D, EH), jnp.float32),
    )(WeT, eaT, be2d)


def _xpose_body(x_ref, out_ref):
    out_ref[...] = x_ref[...].T


def _xpose(x):
    return pl.pallas_call(
        _xpose_body,
        grid=(1,),
        in_specs=[pl.BlockSpec((N, D), lambda i: (0, 0))],
        out_specs=pl.BlockSpec((D, N), lambda i: (0, 0)),
        out_shape=jax.ShapeDtypeStruct((D, N), jnp.float32),
    )(x)


NBUF = 5


NCH = EH // C


def _make_sc_body(ebase, with_init):
    def body(*refs):
        if with_init:
            (xT_hbm, ei_hbm, eT_hbm, init_hbm, out_hbm, xv, av, ev, sv,
             es0, es1, es2, es3, es4, is0, is1, is2, is3, is4, xsem) = refs
        else:
            (xT_hbm, ei_hbm, eT_hbm, out_hbm, xv, av, ev, sv,
             es0, es1, es2, es3, es4, is0, is1, is2, is3, is4, xsem) = refs
            init_hbm = None
        half = lax.axis_index("c")
        pid = lax.axis_index("s")
        wid = pid * NC + half
        esems = (es0, es1, es2, es3, es4)
        isems = (is0, is1, is2, is3, is4)

        xcp = pltpu.make_async_copy(xT_hbm.at[wid], xv, xsem)
        xcp.start()

        def e_copy(ci, b):
            return pltpu.make_async_copy(
                eT_hbm.at[pid, :, pl.ds(ci * C, C)], ev.at[b], esems[b])

        def i_copy(ci, b):
            return pltpu.make_async_copy(
                ei_hbm.at[:, pl.ds(ebase + ci * C, C)], sv.at[b], isems[b])

        for b in range(NBUF):
            e_copy(b, b).start()
            i_copy(b, b).start()

        if with_init:
            xcp.wait()
            pltpu.sync_copy(init_hbm.at[wid], av)
        else:
            zeros = jnp.zeros((L,), jnp.float32)

            @plsc.parallel_loop(0, N // L, unroll=8)
            def _zb(i):
                for d in range(DPT):
                    av[d, pl.ds(i * L, L)] = zeros

            xcp.wait()

        rbase = DPT * half
        dfull = [jnp.full((L,), d, jnp.int32) for d in range(DPT)]

        def outer(j, carry):
            for b in range(NBUF):
                ci = NBUF * j + b
                e_copy(ci, b).wait()
                i_copy(ci, b).wait()

                @plsc.parallel_loop(0, C // L, unroll=4)
                def _gb(g):
                    off = g * L
                    s = sv[b, 0, pl.ds(off, L)]
                    t = sv[b, 1, pl.ds(off, L)]
                    for d in range(DPT):
                        evd = ev[b, rbase + d, pl.ds(off, L)]
                        xg = plsc.load_gather(xv, [dfull[d], s])
                        m = jnp.maximum(xg + evd, 0.0)
                        plsc.addupdate_scatter(av, [dfull[d], t], m)

                nci = ci + NBUF

                @pl.when(nci < NCH)
                def _():
                    e_copy(nci, b).start()
                    i_copy(nci, b).start()
            return carry

        lax.fori_loop(0, NCH // NBUF, outer, 0)
        pltpu.sync_copy(av, out_hbm.at[wid])

    return body


def _sc_aggr(xT, ei, eT_h, phase, init=None):
    mesh = plsc.VectorSubcoreMesh(core_axis_name="c", subcore_axis_name="s")
    scratch = [
        pltpu.VMEM((DPT, N), jnp.float32),
        pltpu.VMEM((DPT, N), jnp.float32),
        pltpu.VMEM((NBUF, 2 * DPT, C), jnp.float32),
        pltpu.VMEM((NBUF, 2, C), jnp.int32),
    ] + [pltpu.SemaphoreType.DMA] * 11
    f = functools.partial(
        pl.kernel,
        out_type=jax.ShapeDtypeStruct((NW, DPT, N), jnp.float32),
        mesh=mesh,
        compiler_params=pltpu.CompilerParams(needs_layout_passes=False),
        scratch_types=scratch,
    )(_make_sc_body(phase * EH, init is not None))
    args = (xT.reshape(NW, DPT, N), ei, eT_h.reshape(NS, NC * DPT, EH))
    if init is not None:
        args = args + (init,)
    return f(*args)


def _mlp_body(x_ref, at_ref, w1_ref, b1_ref, w2_ref, b2_ref, g_ref, bt_ref,
              o_ref):
    x = x_ref[...]
    h = x + at_ref[...].T
    h1 = jnp.maximum(
        jnp.dot(h, w1_ref[...], preferred_element_type=jnp.float32)
        + b1_ref[...], 0.0)
    h2 = jnp.dot(h1, w2_ref[...], preferred_element_type=jnp.float32) \
        + b2_ref[...]
    y = jnp.maximum(h2, 0.0) + x
    mu = jnp.mean(y, axis=-1, keepdims=True)
    yc = y - mu
    var = jnp.mean(yc * yc, axis=-1, keepdims=True)
    o_ref[...] = yc * lax.rsqrt(var + 1e-5) * g_ref[...] + bt_ref[...]


def _mlp(x, aggrT, W1, b1, W2, b2, gamma, beta):
    bn = 1024
    grid = pl.cdiv(N, bn)
    return pl.pallas_call(
        _mlp_body,
        grid=(grid,),
        in_specs=[
            pl.BlockSpec((bn, D), lambda i: (i, 0)),
            pl.BlockSpec((D, bn), lambda i: (0, i)),
            pl.BlockSpec((D, D), lambda i: (0, 0)),
            pl.BlockSpec((1, D), lambda i: (0, 0)),
            pl.BlockSpec((D, D), lambda i: (0, 0)),
            pl.BlockSpec((1, D), lambda i: (0, 0)),
            pl.BlockSpec((1, D), lambda i: (0, 0)),
            pl.BlockSpec((1, D), lambda i: (0, 0)),
        ],
        out_specs=pl.BlockSpec((bn, D), lambda i: (i, 0)),
        out_shape=jax.ShapeDtypeStruct((N, D), jnp.float32),
    )(x, aggrT, W1, b1, W2, b2, gamma, beta)


def kernel(x, edge_index, edge_attr, We, be, W1, b1, W2, b2, gamma, beta):
    ei = edge_index.astype(jnp.int32)
    eaT = edge_attr.T
    WeT = We.T
    be2 = be.reshape(D, 1)
    xT = _xpose(x)
    eT_a = _edge_proj(eaT, WeT, be2, 0)
    aggr_a = _sc_aggr(xT, ei, eT_a, 0)
    eT_b = _edge_proj(eaT, WeT, be2, 1)
    aggr_b = _sc_aggr(xT, ei, eT_b, 1, init=aggr_a)
    return _mlp(x, aggr_b.reshape(D, N), W1, b1.reshape(1, D), W2,
                b2.reshape(1, D), gamma.reshape(1, D), beta.reshape(1, D))


# speedup vs baseline: 1.1559x; 1.0289x over previous
import functools

import jax
import jax.numpy as jnp
from jax import lax
from jax.experimental import pallas as pl
from jax.experimental.pallas import tpu as pltpu
from jax.experimental.pallas import tpu_sc as plsc

N = 10000
E = 320000
D = 128
DE = 16

NC = 2
NS = 16
L = 16
NW = NC * NS
DPT = D // NW
C = 640
NCHUNK = E // C


EH = E // 2


def _edge_proj_body(weT_ref, eaT_ref, be_ref, out_ref):
    e = lax.dot_general(
        weT_ref[...], eaT_ref[...], (((1,), (0,)), ((), ())),
        preferred_element_type=jnp.float32) + be_ref[...]
    out_ref[...] = e.reshape(NW, DPT, e.shape[-1])


def _edge_proj(eaT, WeT, be2d, phase):
    ce = 3200
    grid = EH // ce
    off = phase * (EH // ce)
    return pl.pallas_call(
        _edge_proj_body,
        grid=(grid,),
        in_specs=[
            pl.BlockSpec((D, DE), lambda i: (0, 0)),
            pl.BlockSpec((DE, ce), lambda i: (0, i + off)),
            pl.BlockSpec((D, 1), lambda i: (0, 0)),
        ],
        out_specs=pl.BlockSpec((NW, DPT, ce), lambda i: (0, 0, i)),
        out_shape=jax.ShapeDtypeStruct((NW, DPT, EH), jnp.float32),
    )(WeT, eaT, be2d)


def _xpose_body(x_ref, out_ref):
    out_ref[...] = x_ref[...].T


def _xpose(x):
    return pl.pallas_call(
        _xpose_body,
        grid=(1,),
        in_specs=[pl.BlockSpec((N, D), lambda i: (0, 0))],
        out_specs=pl.BlockSpec((D, N), lambda i: (0, 0)),
        out_shape=jax.ShapeDtypeStruct((D, N), jnp.float32),
    )(x)


NBUF = 5


NCH = EH // C


def _make_sc_body(ebase, with_init):
    def body(*refs):
        if with_init:
            (xT_hbm, ei_hbm, eT_hbm, init_hbm, out_hbm, xv, av, ev, sv,
             es0, es1, es2, es3, es4, is0, is1, is2, is3, is4, xsem) = refs
        else:
            (xT_hbm, ei_hbm, eT_hbm, out_hbm, xv, av, ev, sv,
             es0, es1, es2, es3, es4, is0, is1, is2, is3, is4, xsem) = refs
            init_hbm = None
        half = lax.axis_index("c")
        pid = lax.axis_index("s")
        wid = pid * NC + half
        esems = (es0, es1, es2, es3, es4)
        isems = (is0, is1, is2, is3, is4)

        xcp = pltpu.make_async_copy(xT_hbm.at[wid], xv, xsem)
        xcp.start()

        def e_copy(ci, b):
            return pltpu.make_async_copy(
                eT_hbm.at[wid, :, pl.ds(ci * C, C)], ev.at[b], esems[b])

        def i_copy(ci, b):
            return pltpu.make_async_copy(
                ei_hbm.at[:, pl.ds(ebase + ci * C, C)], sv.at[b], isems[b])

        for b in range(NBUF):
            e_copy(b, b).start()
            i_copy(b, b).start()

        if with_init:
            xcp.wait()
            pltpu.sync_copy(init_hbm.at[wid], av)
        else:
            zeros = jnp.zeros((L,), jnp.float32)

            @plsc.parallel_loop(0, N // L, unroll=8)
            def _zb(i):
                for d in range(DPT):
                    av[d, pl.ds(i * L, L)] = zeros

            xcp.wait()

        dfull = [jnp.full((L,), d, jnp.int32) for d in range(DPT)]

        def outer(j, carry):
            for b in range(NBUF):
                ci = NBUF * j + b
                e_copy(ci, b).wait()
                i_copy(ci, b).wait()

                @plsc.parallel_loop(0, C // L, unroll=4)
                def _gb(g):
                    off = g * L
                    s = sv[b, 0, pl.ds(off, L)]
                    t = sv[b, 1, pl.ds(off, L)]
                    for d in range(DPT):
                        evd = ev[b, d, pl.ds(off, L)]
                        xg = plsc.load_gather(xv, [dfull[d], s])
                        m = jnp.maximum(xg + evd, 0.0)
                        plsc.addupdate_scatter(av, [dfull[d], t], m)

                nci = ci + NBUF

                @pl.when(nci < NCH)
                def _():
                    e_copy(nci, b).start()
                    i_copy(nci, b).start()
            return carry

        lax.fori_loop(0, NCH // NBUF, outer, 0)
        pltpu.sync_copy(av, out_hbm.at[wid])

    return body


def _sc_aggr(xT, ei, eT_h, phase, init=None):
    mesh = plsc.VectorSubcoreMesh(core_axis_name="c", subcore_axis_name="s")
    scratch = [
        pltpu.VMEM((DPT, N), jnp.float32),
        pltpu.VMEM((DPT, N), jnp.float32),
        pltpu.VMEM((NBUF, DPT, C), jnp.float32),
        pltpu.VMEM((NBUF, 2, C), jnp.int32),
    ] + [pltpu.SemaphoreType.DMA] * 11
    f = functools.partial(
        pl.kernel,
        out_type=jax.ShapeDtypeStruct((NW, DPT, N), jnp.float32),
        mesh=mesh,
        compiler_params=pltpu.CompilerParams(needs_layout_passes=False),
        scratch_types=scratch,
    )(_make_sc_body(phase * EH, init is not None))
    args = (xT.reshape(NW, DPT, N), ei, eT_h)
    if init is not None:
        args = args + (init,)
    return f(*args)


def _mlp_body(x_ref, at_ref, w1_ref, b1_ref, w2_ref, b2_ref, g_ref, bt_ref,
              o_ref):
    x = x_ref[...]
    h = x + at_ref[...].T
    h1 = jnp.maximum(
        jnp.dot(h, w1_ref[...], preferred_element_type=jnp.float32)
        + b1_ref[...], 0.0)
    h2 = jnp.dot(h1, w2_ref[...], preferred_element_type=jnp.float32) \
        + b2_ref[...]
    y = jnp.maximum(h2, 0.0) + x
    mu = jnp.mean(y, axis=-1, keepdims=True)
    yc = y - mu
    var = jnp.mean(yc * yc, axis=-1, keepdims=True)
    o_ref[...] = yc * lax.rsqrt(var + 1e-5) * g_ref[...] + bt_ref[...]


def _mlp(x, aggrT, W1, b1, W2, b2, gamma, beta):
    bn = 1024
    grid = pl.cdiv(N, bn)
    return pl.pallas_call(
        _mlp_body,
        grid=(grid,),
        in_specs=[
            pl.BlockSpec((bn, D), lambda i: (i, 0)),
            pl.BlockSpec((D, bn), lambda i: (0, i)),
            pl.BlockSpec((D, D), lambda i: (0, 0)),
            pl.BlockSpec((1, D), lambda i: (0, 0)),
            pl.BlockSpec((D, D), lambda i: (0, 0)),
            pl.BlockSpec((1, D), lambda i: (0, 0)),
            pl.BlockSpec((1, D), lambda i: (0, 0)),
            pl.BlockSpec((1, D), lambda i: (0, 0)),
        ],
        out_specs=pl.BlockSpec((bn, D), lambda i: (i, 0)),
        out_shape=jax.ShapeDtypeStruct((N, D), jnp.float32),
    )(x, aggrT, W1, b1, W2, b2, gamma, beta)


def kernel(x, edge_index, edge_attr, We, be, W1, b1, W2, b2, gamma, beta):
    ei = edge_index.astype(jnp.int32)
    eaT = edge_attr.T
    WeT = We.T
    be2 = be.reshape(D, 1)
    xT = _xpose(x)
    eT_a = _edge_proj(eaT, WeT, be2, 0)
    aggr_a = _sc_aggr(xT, ei, eT_a, 0)
    eT_b = _edge_proj(eaT, WeT, be2, 1)
    aggr_b = _sc_aggr(xT, ei, eT_b, 1, init=aggr_a)
    return _mlp(x, aggr_b.reshape(D, N), W1, b1.reshape(1, D), W2,
                b2.reshape(1, D), gamma.reshape(1, D), beta.reshape(1, D))


# speedup vs baseline: 1.1813x; 1.0220x over previous
import functools

import jax
import jax.numpy as jnp
from jax import lax
from jax.experimental import pallas as pl
from jax.experimental.pallas import tpu as pltpu
from jax.experimental.pallas import tpu_sc as plsc

N = 10000
E = 320000
D = 128
DE = 16

NC = 2
NS = 16
L = 16
NW = NC * NS
DPT = D // NW
C = 1280
NCHUNK = E // C


EH = E // 2


def _edge_proj_body(weT_ref, eaT_ref, be_ref, out_ref):
    e = lax.dot_general(
        weT_ref[...], eaT_ref[...], (((1,), (0,)), ((), ())),
        preferred_element_type=jnp.float32) + be_ref[...]
    out_ref[...] = e.reshape(NW, DPT, e.shape[-1])


def _edge_proj(eaT, WeT, be2d, phase):
    ce = 3200
    grid = EH // ce
    off = phase * (EH // ce)
    return pl.pallas_call(
        _edge_proj_body,
        grid=(grid,),
        in_specs=[
            pl.BlockSpec((D, DE), lambda i: (0, 0)),
            pl.BlockSpec((DE, ce), lambda i: (0, i + off)),
            pl.BlockSpec((D, 1), lambda i: (0, 0)),
        ],
        out_specs=pl.BlockSpec((NW, DPT, ce), lambda i: (0, 0, i)),
        out_shape=jax.ShapeDtypeStruct((NW, DPT, EH), jnp.float32),
    )(WeT, eaT, be2d)


def _xpose_body(x_ref, out_ref):
    out_ref[...] = x_ref[...].T


def _xpose(x):
    return pl.pallas_call(
        _xpose_body,
        grid=(1,),
        in_specs=[pl.BlockSpec((N, D), lambda i: (0, 0))],
        out_specs=pl.BlockSpec((D, N), lambda i: (0, 0)),
        out_shape=jax.ShapeDtypeStruct((D, N), jnp.float32),
    )(x)


NBUF = 5


NCH = EH // C


def _make_sc_body(ebase, with_init):
    def body(*refs):
        if with_init:
            (xT_hbm, ei_hbm, eT_hbm, init_hbm, out_hbm, xv, av, ev, sv,
             es0, es1, es2, es3, es4, is0, is1, is2, is3, is4, xsem) = refs
        else:
            (xT_hbm, ei_hbm, eT_hbm, out_hbm, xv, av, ev, sv,
             es0, es1, es2, es3, es4, is0, is1, is2, is3, is4, xsem) = refs
            init_hbm = None
        half = lax.axis_index("c")
        pid = lax.axis_index("s")
        wid = pid * NC + half
        esems = (es0, es1, es2, es3, es4)
        isems = (is0, is1, is2, is3, is4)

        xcp = pltpu.make_async_copy(xT_hbm.at[wid], xv, xsem)
        xcp.start()

        def e_copy(ci, b):
            return pltpu.make_async_copy(
                eT_hbm.at[wid, :, pl.ds(ci * C, C)], ev.at[b], esems[b])

        def i_copy(ci, b):
            return pltpu.make_async_copy(
                ei_hbm.at[:, pl.ds(ebase + ci * C, C)], sv.at[b], isems[b])

        for b in range(NBUF):
            e_copy(b, b).start()
            i_copy(b, b).start()

        if with_init:
            xcp.wait()
            pltpu.sync_copy(init_hbm.at[wid], av)
        else:
            zeros = jnp.zeros((L,), jnp.float32)

            @plsc.parallel_loop(0, N // L, unroll=8)
            def _zb(i):
                for d in range(DPT):
                    av[d, pl.ds(i * L, L)] = zeros

            xcp.wait()

        dfull = [jnp.full((L,), d, jnp.int32) for d in range(DPT)]

        def outer(j, carry):
            for b in range(NBUF):
                ci = NBUF * j + b
                e_copy(ci, b).wait()
                i_copy(ci, b).wait()

                @plsc.parallel_loop(0, C // L, unroll=4)
                def _gb(g):
                    off = g * L
                    s = sv[b, 0, pl.ds(off, L)]
                    t = sv[b, 1, pl.ds(off, L)]
                    for d in range(DPT):
                        evd = ev[b, d, pl.ds(off, L)]
                        xg = plsc.load_gather(xv, [dfull[d], s])
                        m = jnp.maximum(xg + evd, 0.0)
                        plsc.addupdate_scatter(av, [dfull[d], t], m)

                nci = ci + NBUF

                @pl.when(nci < NCH)
                def _():
                    e_copy(nci, b).start()
                    i_copy(nci, b).start()
            return carry

        lax.fori_loop(0, NCH // NBUF, outer, 0)
        pltpu.sync_copy(av, out_hbm.at[wid])

    return body


def _sc_aggr(xT, ei, eT_h, phase, init=None):
    mesh = plsc.VectorSubcoreMesh(core_axis_name="c", subcore_axis_name="s")
    scratch = [
        pltpu.VMEM((DPT, N), jnp.float32),
        pltpu.VMEM((DPT, N), jnp.float32),
        pltpu.VMEM((NBUF, DPT, C), jnp.float32),
        pltpu.VMEM((NBUF, 2, C), jnp.int32),
    ] + [pltpu.SemaphoreType.DMA] * 11
    f = functools.partial(
        pl.kernel,
        out_type=jax.ShapeDtypeStruct((NW, DPT, N), jnp.float32),
        mesh=mesh,
        compiler_params=pltpu.CompilerParams(needs_layout_passes=False),
        scratch_types=scratch,
    )(_make_sc_body(phase * EH, init is not None))
    args = (xT.reshape(NW, DPT, N), ei, eT_h)
    if init is not None:
        args = args + (init,)
    return f(*args)


def _mlp_body(x_ref, at_ref, w1_ref, b1_ref, w2_ref, b2_ref, g_ref, bt_ref,
              o_ref):
    x = x_ref[...]
    h = x + at_ref[...].T
    h1 = jnp.maximum(
        jnp.dot(h, w1_ref[...], preferred_element_type=jnp.float32)
        + b1_ref[...], 0.0)
    h2 = jnp.dot(h1, w2_ref[...], preferred_element_type=jnp.float32) \
        + b2_ref[...]
    y = jnp.maximum(h2, 0.0) + x
    mu = jnp.mean(y, axis=-1, keepdims=True)
    yc = y - mu
    var = jnp.mean(yc * yc, axis=-1, keepdims=True)
    o_ref[...] = yc * lax.rsqrt(var + 1e-5) * g_ref[...] + bt_ref[...]


def _mlp(x, aggrT, W1, b1, W2, b2, gamma, beta):
    bn = 1024
    grid = pl.cdiv(N, bn)
    return pl.pallas_call(
        _mlp_body,
        grid=(grid,),
        in_specs=[
            pl.BlockSpec((bn, D), lambda i: (i, 0)),
            pl.BlockSpec((D, bn), lambda i: (0, i)),
            pl.BlockSpec((D, D), lambda i: (0, 0)),
            pl.BlockSpec((1, D), lambda i: (0, 0)),
            pl.BlockSpec((D, D), lambda i: (0, 0)),
            pl.BlockSpec((1, D), lambda i: (0, 0)),
            pl.BlockSpec((1, D), lambda i: (0, 0)),
            pl.BlockSpec((1, D), lambda i: (0, 0)),
        ],
        out_specs=pl.BlockSpec((bn, D), lambda i: (i, 0)),
        out_shape=jax.ShapeDtypeStruct((N, D), jnp.float32),
    )(x, aggrT, W1, b1, W2, b2, gamma, beta)


def kernel(x, edge_index, edge_attr, We, be, W1, b1, W2, b2, gamma, beta):
    ei = edge_index.astype(jnp.int32)
    eaT = edge_attr.T
    WeT = We.T
    be2 = be.reshape(D, 1)
    xT = _xpose(x)
    eT_a = _edge_proj(eaT, WeT, be2, 0)
    aggr_a = _sc_aggr(xT, ei, eT_a, 0)
    eT_b = _edge_proj(eaT, WeT, be2, 1)
    aggr_b = _sc_aggr(xT, ei, eT_b, 1, init=aggr_a)
    return _mlp(x, aggr_b.reshape(D, N), W1, b1.reshape(1, D), W2,
                b2.reshape(1, D), gamma.reshape(1, D), beta.reshape(1, D))
